# baseline (speedup 1.0000x reference)
import functools

import jax
import jax.numpy as jnp
from jax import lax
from jax.experimental import pallas as pl
from jax.experimental.pallas import tpu as pltpu
from jax.experimental.pallas import tpu_sc as plsc

N_NODES = 10000
N2 = 10240
HID = 128
NCLS = 10
NG = 64
NC, NS = 2, 16
NW = NC * NS
CH = 128
ROWS_PER = N2 // NS
BR = 2048
GRID = N2 // BR

_MESH = dict(core_axis_name="c", subcore_axis_name="s", num_cores=NC,
             num_subcores=NS)


def _sc_deg(dst2d, z128, ones128, nch):
    mesh = plsc.VectorSubcoreMesh(**_MESH)

    @functools.partial(
        pl.kernel,
        out_type=jax.ShapeDtypeStruct((NC, N2, HID), jnp.float32),
        mesh=mesh,
        scratch_types=[
            pltpu.VMEM((nch, CH), jnp.int32),
            pltpu.VMEM((CH, HID), jnp.float32),
            pltpu.VMEM_SHARED((N2, HID), jnp.float32),
            pltpu.SemaphoreType.DMA,
        ],
    )
    def k(dst_hbm, z_hbm, ones_hbm, out_hbm, didx, onesv, acc, sem):
        c = lax.axis_index("c")
        s = lax.axis_index("s")
        w = s * NC + c
        pltpu.sync_copy(z_hbm.at[pl.ds(s * ROWS_PER, ROWS_PER)],
                        acc.at[pl.ds(s * ROWS_PER, ROWS_PER)])
        pltpu.sync_copy(ones_hbm, onesv)
        pltpu.sync_copy(dst_hbm.at[pl.ds(w * nch, nch)], didx)
        plsc.subcore_barrier()

        fire = 8

        def body(j, carry):
            i = j * fire
            for f in range(fire):
                pltpu.async_copy(onesv, acc.at[didx.at[i + f]], sem, add=True)
            for f in range(fire):
                pltpu.make_async_copy(onesv, acc.at[didx.at[i + f]],
                                      sem).wait()
            return carry

        lax.fori_loop(0, nch // fire, body, 0)
        plsc.subcore_barrier()
        pltpu.sync_copy(acc.at[pl.ds(s * ROWS_PER, ROWS_PER)],
                        out_hbm.at[c, pl.ds(s * ROWS_PER, ROWS_PER)])

    return k(dst2d, z128, ones128)


def _sc_agg(g, src2d, dst2d, z128, nchk):
    mesh = plsc.VectorSubcoreMesh(**_MESH)
    nstg = 4
    half = nchk // nstg
    c2 = 64

    @functools.partial(
        pl.kernel,
        out_type=jax.ShapeDtypeStruct((NC, N2, HID), jnp.float32),
        mesh=mesh,
        scratch_types=[
            pltpu.VMEM((half, c2), jnp.int32),
            pltpu.VMEM((half, c2), jnp.int32),
            pltpu.VMEM((c2, HID), jnp.float32),
            pltpu.VMEM((c2, HID), jnp.float32),
            pltpu.VMEM((c2, HID), jnp.float32),
            pltpu.VMEM((c2, HID), jnp.float32),
            pltpu.VMEM_SHARED((N2, HID), jnp.float32),
            pltpu.SemaphoreType.DMA,
            pltpu.SemaphoreType.DMA,
            pltpu.SemaphoreType.DMA,
            pltpu.SemaphoreType.DMA,
            pltpu.SemaphoreType.DMA,
            pltpu.SemaphoreType.DMA,
            pltpu.SemaphoreType.DMA,
            pltpu.SemaphoreType.DMA,
        ],
    )
    def k(g_hbm, src_hbm, dst_hbm, z_hbm, out_hbm, sidx, didx, b0, b1, b2, b3,
          acc, gs0, gs1, gs2, gs3, ss0, ss1, ss2, ss3):
        bufs = (b0, b1, b2, b3)
        gsem = (gs0, gs1, gs2, gs3)
        ssem = (ss0, ss1, ss2, ss3)
        c = lax.axis_index("c")
        s = lax.axis_index("s")
        w = s * NC + c
        pltpu.sync_copy(z_hbm.at[pl.ds(s * ROWS_PER, ROWS_PER)],
                        acc.at[pl.ds(s * ROWS_PER, ROWS_PER)])
        plsc.subcore_barrier()

        for h in range(nstg):
            base = w * nchk + h * half
            pltpu.sync_copy(src_hbm.at[pl.ds(base, half)], sidx)
            pltpu.sync_copy(dst_hbm.at[pl.ds(base, half)], didx)
            for j in range(3):
                pltpu.async_copy(g_hbm.at[sidx.at[j]], bufs[j], gsem[j])

            def body(k4, carry):
                i0 = 4 * k4
                for j in range(4):
                    i = i0 + j
                    nb = (j + 3) % 4
                    pltpu.make_async_copy(g_hbm.at[sidx.at[i]], bufs[j],
                                          gsem[j]).wait()
                    pltpu.async_copy(bufs[j], acc.at[didx.at[i]], ssem[j],
                                     add=True)

                    @pl.when(jnp.logical_and(i + 3 < half, i >= 1))
                    def _wait_prev_scatter():
                        pltpu.make_async_copy(bufs[nb],
                                              acc.at[didx.at[i - 1]],
                                              ssem[nb]).wait()

                    @pl.when(i + 3 < half)
                    def _fire_gather():
                        pltpu.async_copy(g_hbm.at[sidx.at[i + 3]], bufs[nb],
                                         gsem[nb])
                return carry

            lax.fori_loop(0, half // 4, body, 0)
            for j in range(4):
                pltpu.make_async_copy(bufs[j], acc.at[didx.at[half - 4 + j]],
                                      ssem[j]).wait()
        plsc.subcore_barrier()
        pltpu.sync_copy(acc.at[pl.ds(s * ROWS_PER, ROWS_PER)],
                        out_hbm.at[c, pl.ds(s * ROWS_PER, ROWS_PER)])

    return k(g, src2d, dst2d, z128)


def _prep0_body(dp_ref, x_ref, w_ref, g_ref, dinv_ref):
    deg = dp_ref[0, :, 0:1] + dp_ref[1, :, 0:1] + 1.0
    dinv = lax.rsqrt(deg)
    h = jnp.dot(x_ref[...], w_ref[...], preferred_element_type=jnp.float32)
    g_ref[...] = h * dinv
    dinv_ref[...] = dinv


def _prep0(dp, xp, W0):
    return pl.pallas_call(
        _prep0_body,
        grid=(GRID,),
        in_specs=[
            pl.BlockSpec((2, BR, HID), lambda i: (0, i, 0)),
            pl.BlockSpec((BR, HID), lambda i: (i, 0)),
            pl.BlockSpec((HID, HID), lambda i: (0, 0)),
        ],
        out_specs=[
            pl.BlockSpec((BR, HID), lambda i: (i, 0)),
            pl.BlockSpec((BR, 1), lambda i: (i, 0)),
        ],
        out_shape=[
            jax.ShapeDtypeStruct((N2, HID), jnp.float32),
            jax.ShapeDtypeStruct((N2, 1), jnp.float32),
        ],
    )(dp, xp, W0)


def _layer_body(p_ref, g_ref, dinv_ref, b_ref, w_ref, gn_ref):
    out = (p_ref[0] + p_ref[1] + g_ref[...]) * dinv_ref[...] + b_ref[...]
    h = jnp.maximum(out, 0.0)
    gn_ref[...] = jnp.dot(h, w_ref[...],
                          preferred_element_type=jnp.float32) * dinv_ref[...]


def _layer(P, g, dinv, b_row, W_next):
    return pl.pallas_call(
        _layer_body,
        grid=(GRID,),
        in_specs=[
            pl.BlockSpec((2, BR, HID), lambda i: (0, i, 0)),
            pl.BlockSpec((BR, HID), lambda i: (i, 0)),
            pl.BlockSpec((BR, 1), lambda i: (i, 0)),
            pl.BlockSpec((1, HID), lambda i: (0, 0)),
            pl.BlockSpec((HID, HID), lambda i: (0, 0)),
        ],
        out_specs=pl.BlockSpec((BR, HID), lambda i: (i, 0)),
        out_shape=jax.ShapeDtypeStruct((N2, HID), jnp.float32),
    )(P, g, dinv, b_row, W_next)


def _final_body(p_ref, g_ref, dinv_ref, b_ref, bat_ref, lw_ref, lb_ref,
                out_ref, acc, cnt):
    i = pl.program_id(0)

    @pl.when(i == 0)
    def _init():
        acc[...] = jnp.zeros_like(acc)
        cnt[...] = jnp.zeros_like(cnt)

    h = (p_ref[0] + p_ref[1] + g_ref[...]) * dinv_ref[...] + b_ref[...]
    m = (bat_ref[...] == lax.broadcasted_iota(jnp.int32, (NG, BR), 0)
         ).astype(jnp.float32)
    acc[...] += jnp.dot(m, h, preferred_element_type=jnp.float32)
    cnt[...] += jnp.sum(m, axis=1, keepdims=True)

    @pl.when(i == pl.num_programs(0) - 1)
    def _fin():
        pooled = acc[...] / jnp.maximum(cnt[...], 1.0)
        logits = jnp.dot(pooled, lw_ref[...],
                         preferred_element_type=jnp.float32) + lb_ref[...]
        z = logits - jnp.max(logits, axis=1, keepdims=True)
        e = jnp.exp(z)
        out_ref[...] = e / jnp.sum(e, axis=1, keepdims=True)


def _final(R, g2, dinv, b_row, batp, lin_W, lb_row):
    return pl.pallas_call(
        _final_body,
        grid=(GRID,),
        in_specs=[
            pl.BlockSpec((2, BR, HID), lambda i: (0, i, 0)),
            pl.BlockSpec((BR, HID), lambda i: (i, 0)),
            pl.BlockSpec((BR, 1), lambda i: (i, 0)),
            pl.BlockSpec((1, HID), lambda i: (0, 0)),
            pl.BlockSpec((1, BR), lambda i: (0, i)),
            pl.BlockSpec((HID, NCLS), lambda i: (0, 0)),
            pl.BlockSpec((1, NCLS), lambda i: (0, 0)),
        ],
        out_specs=pl.BlockSpec((NG, NCLS), lambda i: (0, 0)),
        out_shape=jax.ShapeDtypeStruct((NG, NCLS), jnp.float32),
        scratch_shapes=[
            pltpu.VMEM((NG, HID), jnp.float32),
            pltpu.VMEM((NG, 1), jnp.float32),
        ],
    )(R, g2, dinv, b_row, batp, lin_W, lb_row)


def kernel(x, edge_index, batch, W0, b0, W1, b1, W2, b2, lin_W, lin_b):
    E = edge_index.shape[1]
    nch = -(-E // (NW * CH))
    nch = -(-nch // 8) * 8
    e_pad = nch * NW * CH
    pad_e = e_pad - E
    pad_ids = N_NODES + jnp.arange(pad_e, dtype=jnp.int32) % (N2 - N_NODES)
    src = jnp.concatenate([edge_index[0], pad_ids])
    dst = jnp.concatenate([edge_index[1], pad_ids])
    src2d64 = src.reshape(e_pad // 64, 64)
    dst2d64 = dst.reshape(e_pad // 64, 64)
    dst2d = dst.reshape(e_pad // CH, CH)
    nchk = 2 * nch
    xp = jnp.zeros((N2, HID), jnp.float32).at[:N_NODES].set(x)
    batp = jnp.full((1, N2), NG, jnp.int32).at[0, :N_NODES].set(batch)
    z128 = jnp.zeros((N2, HID), jnp.float32)
    ones128 = jnp.ones((CH, HID), jnp.float32)

    dp = _sc_deg(dst2d, z128, ones128, nch)
    g0, dinv = _prep0(dp, xp, W0)
    P = _sc_agg(g0, src2d64, dst2d64, z128, nchk)
    g1 = _layer(P, g0, dinv, b0.reshape(1, HID), W1)
    Q = _sc_agg(g1, src2d64, dst2d64, z128, nchk)
    g2 = _layer(Q, g1, dinv, b1.reshape(1, HID), W2)
    R = _sc_agg(g2, src2d64, dst2d64, z128, nchk)
    return _final(R, g2, dinv, b2.reshape(1, HID), batp, lin_W,
                  lin_b.reshape(1, NCLS))


# speedup vs baseline: 1.0015x; 1.0015x over previous
import functools

import jax
import jax.numpy as jnp
from jax import lax
from jax.experimental import pallas as pl
from jax.experimental.pallas import tpu as pltpu
from jax.experimental.pallas import tpu_sc as plsc

N_NODES = 10000
N2 = 10240
HID = 128
NCLS = 10
NG = 64
NC, NS = 2, 16
NW = NC * NS
CH = 128
ROWS_PER = N2 // NS
BR = 2048
GRID = N2 // BR

_MESH = dict(core_axis_name="c", subcore_axis_name="s", num_cores=NC,
             num_subcores=NS)


def _sc_deg(dst2d, z128, ones128, nch):
    mesh = plsc.VectorSubcoreMesh(**_MESH)

    @functools.partial(
        pl.kernel,
        out_type=jax.ShapeDtypeStruct((NC, N2, HID), jnp.float32),
        mesh=mesh,
        scratch_types=[
            pltpu.VMEM((nch, CH), jnp.int32),
            pltpu.VMEM((CH, HID), jnp.float32),
            pltpu.VMEM_SHARED((N2, HID), jnp.float32),
            pltpu.SemaphoreType.DMA,
        ],
    )
    def k(dst_hbm, z_hbm, ones_hbm, out_hbm, didx, onesv, acc, sem):
        c = lax.axis_index("c")
        s = lax.axis_index("s")
        w = s * NC + c
        pltpu.sync_copy(z_hbm.at[pl.ds(s * ROWS_PER, ROWS_PER)],
                        acc.at[pl.ds(s * ROWS_PER, ROWS_PER)])
        pltpu.sync_copy(ones_hbm, onesv)
        pltpu.sync_copy(dst_hbm.at[pl.ds(w * nch, nch)], didx)
        plsc.subcore_barrier()

        fire = 8

        def body(j, carry):
            i = j * fire
            for f in range(fire):
                pltpu.async_copy(onesv, acc.at[didx.at[i + f]], sem, add=True)
            for f in range(fire):
                pltpu.make_async_copy(onesv, acc.at[didx.at[i + f]],
                                      sem).wait()
            return carry

        lax.fori_loop(0, nch // fire, body, 0)
        plsc.subcore_barrier()
        pltpu.sync_copy(acc.at[pl.ds(s * ROWS_PER, ROWS_PER)],
                        out_hbm.at[c, pl.ds(s * ROWS_PER, ROWS_PER)])

    return k(dst2d, z128, ones128)


def _sc_agg(g, src2d, dst2d, z128, nchk):
    mesh = plsc.VectorSubcoreMesh(**_MESH)
    nstg = 4
    half = nchk // nstg
    c2 = 64

    @functools.partial(
        pl.kernel,
        out_type=jax.ShapeDtypeStruct((NC, N2, HID), jnp.float32),
        mesh=mesh,
        scratch_types=[
            pltpu.VMEM((half, c2), jnp.int32),
            pltpu.VMEM((half, c2), jnp.int32),
            pltpu.VMEM((c2, HID), jnp.float32),
            pltpu.VMEM((c2, HID), jnp.float32),
            pltpu.VMEM((c2, HID), jnp.float32),
            pltpu.VMEM((c2, HID), jnp.float32),
            pltpu.VMEM_SHARED((N2, HID), jnp.float32),
            pltpu.SemaphoreType.DMA,
            pltpu.SemaphoreType.DMA,
            pltpu.SemaphoreType.DMA,
            pltpu.SemaphoreType.DMA,
            pltpu.SemaphoreType.DMA,
            pltpu.SemaphoreType.DMA,
            pltpu.SemaphoreType.DMA,
            pltpu.SemaphoreType.DMA,
        ],
    )
    def k(g_hbm, src_hbm, dst_hbm, z_hbm, out_hbm, sidx, didx, b0, b1, b2, b3,
          acc, gs0, gs1, gs2, gs3, ss0, ss1, ss2, ss3):
        bufs = (b0, b1, b2, b3)
        gsem = (gs0, gs1, gs2, gs3)
        ssem = (ss0, ss1, ss2, ss3)
        c = lax.axis_index("c")
        s = lax.axis_index("s")
        w = s * NC + c
        pltpu.sync_copy(z_hbm.at[pl.ds(s * ROWS_PER, ROWS_PER)],
                        acc.at[pl.ds(s * ROWS_PER, ROWS_PER)])
        plsc.subcore_barrier()

        for h in range(nstg):
            base = w * nchk + h * half
            pltpu.sync_copy(src_hbm.at[pl.ds(base, half)], sidx)
            pltpu.sync_copy(dst_hbm.at[pl.ds(base, half)], didx)
            for j in range(3):
                pltpu.async_copy(g_hbm.at[sidx.at[j]], bufs[j], gsem[j])

            def body(k4, carry):
                i0 = 4 * k4
                for j in range(4):
                    i = i0 + j
                    nb = (j + 3) % 4
                    pltpu.make_async_copy(g_hbm.at[sidx.at[i]], bufs[j],
                                          gsem[j]).wait()
                    pltpu.async_copy(bufs[j], acc.at[didx.at[i]], ssem[j],
                                     add=True)

                    @pl.when(jnp.logical_and(i + 3 < half, i >= 1))
                    def _wait_prev_scatter():
                        pltpu.make_async_copy(bufs[nb],
                                              acc.at[didx.at[i - 1]],
                                              ssem[nb]).wait()

                    @pl.when(i + 3 < half)
                    def _fire_gather():
                        pltpu.async_copy(g_hbm.at[sidx.at[i + 3]], bufs[nb],
                                         gsem[nb])
                return carry

            lax.fori_loop(0, half // 4, body, 0)
            for j in range(4):
                pltpu.make_async_copy(bufs[j], acc.at[didx.at[half - 4 + j]],
                                      ssem[j]).wait()
        plsc.subcore_barrier()
        pltpu.sync_copy(acc.at[pl.ds(s * ROWS_PER, ROWS_PER)],
                        out_hbm.at[c, pl.ds(s * ROWS_PER, ROWS_PER)])

    return k(g, src2d, dst2d, z128)


def _mm0_body(x_ref, w_ref, h_ref):
    h_ref[...] = jnp.dot(x_ref[...], w_ref[...],
                         preferred_element_type=jnp.float32)


def _mm0(xp, W0):
    return pl.pallas_call(
        _mm0_body,
        grid=(GRID,),
        in_specs=[
            pl.BlockSpec((BR, HID), lambda i: (i, 0)),
            pl.BlockSpec((HID, HID), lambda i: (0, 0)),
        ],
        out_specs=pl.BlockSpec((BR, HID), lambda i: (i, 0)),
        out_shape=jax.ShapeDtypeStruct((N2, HID), jnp.float32),
    )(xp, W0)


def _scale0_body(dp_ref, h_ref, g_ref, dinv_ref):
    deg = dp_ref[0, :, 0:1] + dp_ref[1, :, 0:1] + 1.0
    dinv = lax.rsqrt(deg)
    g_ref[...] = h_ref[...] * dinv
    dinv_ref[...] = dinv


def _scale0(dp, h0):
    return pl.pallas_call(
        _scale0_body,
        grid=(GRID,),
        in_specs=[
            pl.BlockSpec((2, BR, HID), lambda i: (0, i, 0)),
            pl.BlockSpec((BR, HID), lambda i: (i, 0)),
        ],
        out_specs=[
            pl.BlockSpec((BR, HID), lambda i: (i, 0)),
            pl.BlockSpec((BR, 1), lambda i: (i, 0)),
        ],
        out_shape=[
            jax.ShapeDtypeStruct((N2, HID), jnp.float32),
            jax.ShapeDtypeStruct((N2, 1), jnp.float32),
        ],
    )(dp, h0)


def _layer_body(p_ref, g_ref, dinv_ref, b_ref, w_ref, gn_ref):
    out = (p_ref[0] + p_ref[1] + g_ref[...]) * dinv_ref[...] + b_ref[...]
    h = jnp.maximum(out, 0.0)
    gn_ref[...] = jnp.dot(h, w_ref[...],
                          preferred_element_type=jnp.float32) * dinv_ref[...]


def _layer(P, g, dinv, b_row, W_next):
    return pl.pallas_call(
        _layer_body,
        grid=(GRID,),
        in_specs=[
            pl.BlockSpec((2, BR, HID), lambda i: (0, i, 0)),
            pl.BlockSpec((BR, HID), lambda i: (i, 0)),
            pl.BlockSpec((BR, 1), lambda i: (i, 0)),
            pl.BlockSpec((1, HID), lambda i: (0, 0)),
            pl.BlockSpec((HID, HID), lambda i: (0, 0)),
        ],
        out_specs=pl.BlockSpec((BR, HID), lambda i: (i, 0)),
        out_shape=jax.ShapeDtypeStruct((N2, HID), jnp.float32),
    )(P, g, dinv, b_row, W_next)


def _final_body(p_ref, g_ref, dinv_ref, b_ref, bat_ref, lw_ref, lb_ref,
                out_ref, acc, cnt):
    i = pl.program_id(0)

    @pl.when(i == 0)
    def _init():
        acc[...] = jnp.zeros_like(acc)
        cnt[...] = jnp.zeros_like(cnt)

    h = (p_ref[0] + p_ref[1] + g_ref[...]) * dinv_ref[...] + b_ref[...]
    m = (bat_ref[...] == lax.broadcasted_iota(jnp.int32, (NG, BR), 0)
         ).astype(jnp.float32)
    acc[...] += jnp.dot(m, h, preferred_element_type=jnp.float32)
    cnt[...] += jnp.sum(m, axis=1, keepdims=True)

    @pl.when(i == pl.num_programs(0) - 1)
    def _fin():
        pooled = acc[...] / jnp.maximum(cnt[...], 1.0)
        logits = jnp.dot(pooled, lw_ref[...],
                         preferred_element_type=jnp.float32) + lb_ref[...]
        z = logits - jnp.max(logits, axis=1, keepdims=True)
        e = jnp.exp(z)
        out_ref[...] = e / jnp.sum(e, axis=1, keepdims=True)


def _final(R, g2, dinv, b_row, batp, lin_W, lb_row):
    return pl.pallas_call(
        _final_body,
        grid=(GRID,),
        in_specs=[
            pl.BlockSpec((2, BR, HID), lambda i: (0, i, 0)),
            pl.BlockSpec((BR, HID), lambda i: (i, 0)),
            pl.BlockSpec((BR, 1), lambda i: (i, 0)),
            pl.BlockSpec((1, HID), lambda i: (0, 0)),
            pl.BlockSpec((1, BR), lambda i: (0, i)),
            pl.BlockSpec((HID, NCLS), lambda i: (0, 0)),
            pl.BlockSpec((1, NCLS), lambda i: (0, 0)),
        ],
        out_specs=pl.BlockSpec((NG, NCLS), lambda i: (0, 0)),
        out_shape=jax.ShapeDtypeStruct((NG, NCLS), jnp.float32),
        scratch_shapes=[
            pltpu.VMEM((NG, HID), jnp.float32),
            pltpu.VMEM((NG, 1), jnp.float32),
        ],
    )(R, g2, dinv, b_row, batp, lin_W, lb_row)


def kernel(x, edge_index, batch, W0, b0, W1, b1, W2, b2, lin_W, lin_b):
    E = edge_index.shape[1]
    nch = -(-E // (NW * CH))
    nch = -(-nch // 8) * 8
    e_pad = nch * NW * CH
    pad_e = e_pad - E
    pad_ids = N_NODES + jnp.arange(pad_e, dtype=jnp.int32) % (N2 - N_NODES)
    src = jnp.concatenate([edge_index[0], pad_ids])
    dst = jnp.concatenate([edge_index[1], pad_ids])
    src2d64 = src.reshape(e_pad // 64, 64)
    dst2d64 = dst.reshape(e_pad // 64, 64)
    dst2d = dst.reshape(e_pad // CH, CH)
    nchk = 2 * nch
    xp = jnp.zeros((N2, HID), jnp.float32).at[:N_NODES].set(x)
    batp = jnp.full((1, N2), NG, jnp.int32).at[0, :N_NODES].set(batch)
    z128 = jnp.zeros((N2, HID), jnp.float32)
    ones128 = jnp.ones((CH, HID), jnp.float32)

    h0 = _mm0(xp, W0)
    dp = _sc_deg(dst2d, z128, ones128, nch)
    g0, dinv = _scale0(dp, h0)
    P = _sc_agg(g0, src2d64, dst2d64, z128, nchk)
    g1 = _layer(P, g0, dinv, b0.reshape(1, HID), W1)
    Q = _sc_agg(g1, src2d64, dst2d64, z128, nchk)
    g2 = _layer(Q, g1, dinv, b1.reshape(1, HID), W2)
    R = _sc_agg(g2, src2d64, dst2d64, z128, nchk)
    return _final(R, g2, dinv, b2.reshape(1, HID), batp, lin_W,
                  lin_b.reshape(1, NCLS))


# speedup vs baseline: 1.0241x; 1.0226x over previous
import functools

import jax
import jax.numpy as jnp
from jax import lax
from jax.experimental import pallas as pl
from jax.experimental.pallas import tpu as pltpu
from jax.experimental.pallas import tpu_sc as plsc

N_NODES = 10000
N2 = 10240
HID = 128
NCLS = 10
NG = 64
NC, NS = 2, 16
NW = NC * NS
CH = 128
ROWS_PER = N2 // NS
BR = 2048
GRID = N2 // BR

_MESH = dict(core_axis_name="c", subcore_axis_name="s", num_cores=NC,
             num_subcores=NS)


def _sc_deg(dst2d, z128, ones128, nch):
    mesh = plsc.VectorSubcoreMesh(**_MESH)

    @functools.partial(
        pl.kernel,
        out_type=jax.ShapeDtypeStruct((NC, N2, HID), jnp.float32),
        mesh=mesh,
        scratch_types=[
            pltpu.VMEM((nch, CH), jnp.int32),
            pltpu.VMEM((CH, HID), jnp.float32),
            pltpu.VMEM_SHARED((N2, HID), jnp.float32),
        ],
    )
    def k(dst_hbm, z_hbm, ones_hbm, out_hbm, didx, onesv, acc):
        c = lax.axis_index("c")
        s = lax.axis_index("s")
        w = s * NC + c
        pltpu.sync_copy(z_hbm.at[pl.ds(s * ROWS_PER, ROWS_PER)],
                        acc.at[pl.ds(s * ROWS_PER, ROWS_PER)])
        pltpu.sync_copy(ones_hbm, onesv)
        pltpu.sync_copy(dst_hbm.at[pl.ds(w * nch, nch)], didx)
        plsc.subcore_barrier()

        def body(i, carry):
            pltpu.sync_copy(onesv, acc.at[didx.at[i]], add=True)
            return carry

        lax.fori_loop(0, nch, body, 0)
        plsc.subcore_barrier()
        pltpu.sync_copy(acc.at[pl.ds(s * ROWS_PER, ROWS_PER)],
                        out_hbm.at[c, pl.ds(s * ROWS_PER, ROWS_PER)])

    return k(dst2d, z128, ones128)


def _sc_agg(g, src2d, dst2d, z128, nch):
    mesh = plsc.VectorSubcoreMesh(**_MESH)

    @functools.partial(
        pl.kernel,
        out_type=jax.ShapeDtypeStruct((NC, N2, HID), jnp.float32),
        mesh=mesh,
        scratch_types=[
            pltpu.VMEM((nch // 2, CH), jnp.int32),
            pltpu.VMEM((nch // 2, CH), jnp.int32),
            pltpu.VMEM((CH, HID), jnp.float32),
            pltpu.VMEM((CH, HID), jnp.float32),
            pltpu.VMEM_SHARED((N2, HID), jnp.float32),
            pltpu.SemaphoreType.DMA,
            pltpu.SemaphoreType.DMA,
        ],
    )
    def k(g_hbm, src_hbm, dst_hbm, z_hbm, out_hbm, sidx, didx, buf_a, buf_b,
          acc, sem_a, sem_b):
        c = lax.axis_index("c")
        s = lax.axis_index("s")
        w = s * NC + c
        half = nch // 2
        pltpu.sync_copy(z_hbm.at[pl.ds(s * ROWS_PER, ROWS_PER)],
                        acc.at[pl.ds(s * ROWS_PER, ROWS_PER)])
        plsc.subcore_barrier()

        for h in range(2):
            pltpu.sync_copy(src_hbm.at[pl.ds(w * nch + h * half, half)], sidx)
            pltpu.sync_copy(dst_hbm.at[pl.ds(w * nch + h * half, half)], didx)
            pltpu.async_copy(g_hbm.at[sidx.at[0]], buf_a, sem_a)

            def body(k2, carry):
                i = 2 * k2
                pltpu.async_copy(g_hbm.at[sidx.at[i + 1]], buf_b, sem_b)
                pltpu.make_async_copy(g_hbm.at[sidx.at[i]], buf_a,
                                      sem_a).wait()
                pltpu.sync_copy(buf_a, acc.at[didx.at[i]], add=True)

                @pl.when(i + 2 < half)
                def _next():
                    pltpu.async_copy(g_hbm.at[sidx.at[i + 2]], buf_a, sem_a)

                pltpu.make_async_copy(g_hbm.at[sidx.at[i + 1]], buf_b,
                                      sem_b).wait()
                pltpu.sync_copy(buf_b, acc.at[didx.at[i + 1]], add=True)
                return carry

            lax.fori_loop(0, half // 2, body, 0)
        plsc.subcore_barrier()
        pltpu.sync_copy(acc.at[pl.ds(s * ROWS_PER, ROWS_PER)],
                        out_hbm.at[c, pl.ds(s * ROWS_PER, ROWS_PER)])

    return k(g, src2d, dst2d, z128)


def _prep0_body(dp_ref, x_ref, w_ref, g_ref, dinv_ref):
    deg = dp_ref[0, :, 0:1] + dp_ref[1, :, 0:1] + 1.0
    dinv = lax.rsqrt(deg)
    h = jnp.dot(x_ref[...], w_ref[...], preferred_element_type=jnp.float32)
    g_ref[...] = h * dinv
    dinv_ref[...] = dinv


def _prep0(dp, xp, W0):
    return pl.pallas_call(
        _prep0_body,
        grid=(GRID,),
        in_specs=[
            pl.BlockSpec((2, BR, HID), lambda i: (0, i, 0)),
            pl.BlockSpec((BR, HID), lambda i: (i, 0)),
            pl.BlockSpec((HID, HID), lambda i: (0, 0)),
        ],
        out_specs=[
            pl.BlockSpec((BR, HID), lambda i: (i, 0)),
            pl.BlockSpec((BR, 1), lambda i: (i, 0)),
        ],
        out_shape=[
            jax.ShapeDtypeStruct((N2, HID), jnp.float32),
            jax.ShapeDtypeStruct((N2, 1), jnp.float32),
        ],
    )(dp, xp, W0)


def _layer_body(p_ref, g_ref, dinv_ref, b_ref, w_ref, gn_ref):
    out = (p_ref[0] + p_ref[1] + g_ref[...]) * dinv_ref[...] + b_ref[...]
    h = jnp.maximum(out, 0.0)
    gn_ref[...] = jnp.dot(h, w_ref[...],
                          preferred_element_type=jnp.float32) * dinv_ref[...]


def _layer(P, g, dinv, b_row, W_next):
    return pl.pallas_call(
        _layer_body,
        grid=(GRID,),
        in_specs=[
            pl.BlockSpec((2, BR, HID), lambda i: (0, i, 0)),
            pl.BlockSpec((BR, HID), lambda i: (i, 0)),
            pl.BlockSpec((BR, 1), lambda i: (i, 0)),
            pl.BlockSpec((1, HID), lambda i: (0, 0)),
            pl.BlockSpec((HID, HID), lambda i: (0, 0)),
        ],
        out_specs=pl.BlockSpec((BR, HID), lambda i: (i, 0)),
        out_shape=jax.ShapeDtypeStruct((N2, HID), jnp.float32),
    )(P, g, dinv, b_row, W_next)


def _final_body(p_ref, g_ref, dinv_ref, b_ref, bat_ref, lw_ref, lb_ref,
                out_ref, acc, cnt):
    i = pl.program_id(0)

    @pl.when(i == 0)
    def _init():
        acc[...] = jnp.zeros_like(acc)
        cnt[...] = jnp.zeros_like(cnt)

    h = (p_ref[0] + p_ref[1] + g_ref[...]) * dinv_ref[...] + b_ref[...]
    m = (bat_ref[...] == lax.broadcasted_iota(jnp.int32, (NG, BR), 0)
         ).astype(jnp.float32)
    acc[...] += jnp.dot(m, h, preferred_element_type=jnp.float32)
    cnt[...] += jnp.sum(m, axis=1, keepdims=True)

    @pl.when(i == pl.num_programs(0) - 1)
    def _fin():
        pooled = acc[...] / jnp.maximum(cnt[...], 1.0)
        logits = jnp.dot(pooled, lw_ref[...],
                         preferred_element_type=jnp.float32) + lb_ref[...]
        z = logits - jnp.max(logits, axis=1, keepdims=True)
        e = jnp.exp(z)
        out_ref[...] = e / jnp.sum(e, axis=1, keepdims=True)


def _final(R, g2, dinv, b_row, batp, lin_W, lb_row):
    return pl.pallas_call(
        _final_body,
        grid=(GRID,),
        in_specs=[
            pl.BlockSpec((2, BR, HID), lambda i: (0, i, 0)),
            pl.BlockSpec((BR, HID), lambda i: (i, 0)),
            pl.BlockSpec((BR, 1), lambda i: (i, 0)),
            pl.BlockSpec((1, HID), lambda i: (0, 0)),
            pl.BlockSpec((1, BR), lambda i: (0, i)),
            pl.BlockSpec((HID, NCLS), lambda i: (0, 0)),
            pl.BlockSpec((1, NCLS), lambda i: (0, 0)),
        ],
        out_specs=pl.BlockSpec((NG, NCLS), lambda i: (0, 0)),
        out_shape=jax.ShapeDtypeStruct((NG, NCLS), jnp.float32),
        scratch_shapes=[
            pltpu.VMEM((NG, HID), jnp.float32),
            pltpu.VMEM((NG, 1), jnp.float32),
        ],
    )(R, g2, dinv, b_row, batp, lin_W, lb_row)


def kernel(x, edge_index, batch, W0, b0, W1, b1, W2, b2, lin_W, lin_b):
    E = edge_index.shape[1]
    nch = -(-E // (NW * CH))
    nch = -(-nch // 8) * 8
    e_pad = nch * NW * CH
    pad_e = e_pad - E
    pad_ids = N_NODES + jnp.arange(pad_e, dtype=jnp.int32) % (N2 - N_NODES)
    src = jnp.concatenate([edge_index[0], pad_ids])
    dst = jnp.concatenate([edge_index[1], pad_ids])
    src2d = src.reshape(e_pad // CH, CH)
    dst2d = dst.reshape(e_pad // CH, CH)
    xp = jnp.zeros((N2, HID), jnp.float32).at[:N_NODES].set(x)
    batp = jnp.full((1, N2), NG, jnp.int32).at[0, :N_NODES].set(batch)
    z128 = jnp.zeros((N2, HID), jnp.float32)
    ones128 = jnp.ones((CH, HID), jnp.float32)

    dp = _sc_deg(dst2d, z128, ones128, nch)
    g0, dinv = _prep0(dp, xp, W0)
    P = _sc_agg(g0, src2d, dst2d, z128, nch)
    g1 = _layer(P, g0, dinv, b0.reshape(1, HID), W1)
    Q = _sc_agg(g1, src2d, dst2d, z128, nch)
    g2 = _layer(Q, g1, dinv, b1.reshape(1, HID), W2)
    R = _sc_agg(g2, src2d, dst2d, z128, nch)
    return _final(R, g2, dinv, b2.reshape(1, HID), batp, lin_W,
                  lin_b.reshape(1, NCLS))
